# baseline (speedup 1.0000x reference)
import jax
import jax.numpy as jnp
from jax import lax
from jax.experimental import pallas as pl
from jax.experimental.pallas import tpu as pltpu
from jax.experimental.pallas import tpu_sc as plsc

VOCAB = 1_000_000
DIM = 64
BATCH = 4096
HALF = BATCH // 2
HIST = 200
N = BATCH * HIST

NC, NS = 2, 16
NW = NC * NS
PAIRS_PER_W = HALF // NW
ROWS_PER_W = N // NW
HALF_SLAB = PAIRS_PER_W * HIST
NBANK = 2
STEPS = PAIRS_PER_W - NBANK


def _gather_body(idx_hbm, table_hbm, out_hbm, idx_v, rows_v, gsems, ssems):
    c = lax.axis_index("c")
    s = lax.axis_index("s")
    wid = s * NC + c
    pair0 = wid * PAIRS_PER_W

    pltpu.sync_copy(idx_hbm.at[0, wid], idx_v.at[pl.ds(0, HALF_SLAB)])
    pltpu.sync_copy(idx_hbm.at[1, wid], idx_v.at[pl.ds(HALF_SLAB, HALF_SLAB)])

    def gather(p, g, half):
        return pltpu.make_async_copy(
            table_hbm.at[idx_v.at[pl.ds(half * HALF_SLAB + g * HIST, HIST)]],
            rows_v.at[p, half], gsems.at[p, half])

    def store(p, g, half):
        return pltpu.make_async_copy(
            rows_v.at[p, half],
            out_hbm.at[pair0 + g, :, pl.ds(half * DIM, DIM)],
            ssems.at[p, half])

    for p in range(NBANK):
        gather(p, p, 0).start()
        gather(p, p, 1).start()

    def step(g, _):
        for p in range(NBANK):
            grp = g * NBANK + p
            for half in range(2):
                gather(p, grp, half).wait()
                store(p, grp, half).start()
            for half in range(2):
                store(p, grp, half).wait()
                gather(p, grp + NBANK, half).start()
        return _

    lax.fori_loop(0, STEPS // NBANK, step, None)

    for p in range(NBANK):
        grp = STEPS + p
        for half in range(2):
            gather(p, grp, half).wait()
            store(p, grp, half).start()
    for p in range(NBANK):
        for half in range(2):
            store(p, STEPS + p, half).wait()


@jax.jit
def _sc_gather(idx, table):
    mesh = plsc.VectorSubcoreMesh(
        core_axis_name="c", subcore_axis_name="s",
        num_cores=NC, num_subcores=NS)
    k = pl.kernel(
        _gather_body,
        out_type=jax.ShapeDtypeStruct((HALF, HIST, 2 * DIM), jnp.float32),
        mesh=mesh,
        compiler_params=pltpu.CompilerParams(use_tc_tiling_on_sc=False),
        scratch_types=[
            pltpu.VMEM((ROWS_PER_W,), jnp.int32),
            pltpu.VMEM((NBANK, 2, HIST, DIM), jnp.float32),
            pltpu.SemaphoreType.DMA((NBANK, 2)),
            pltpu.SemaphoreType.DMA((NBANK, 2)),
        ],
    )
    return k(idx, table)


HCH = 8


def _t2_body(in_ref, out_ref):
    def body(hh, _):
        t = in_ref[:, hh, :]
        tt = t.T
        v = tt.reshape(2, DIM, HALF).transpose(1, 0, 2)
        out_ref[hh] = v.reshape(DIM, BATCH)
        return _
    lax.fori_loop(0, HCH, body, None)


@jax.jit
def _to_out_layout(pair_rows):
    return pl.pallas_call(
        _t2_body,
        grid=(HIST // HCH,),
        in_specs=[pl.BlockSpec((HALF, HCH, 2 * DIM), lambda i: (0, i, 0))],
        out_specs=pl.BlockSpec((HCH, DIM, BATCH), lambda i: (i, 0, 0)),
        out_shape=jax.ShapeDtypeStruct((HIST, DIM, BATCH), jnp.float32),
    )(pair_rows)


VCH = 8000


def _t1_body(in_ref, out_ref):
    t3 = in_ref[...].reshape(VCH // 2, 2, DIM)
    out_ref[:, :DIM] = t3[:, 0, :]
    out_ref[:, DIM:] = t3[:, 1, :]


@jax.jit
def _to_table_layout(w):
    return pl.pallas_call(
        _t1_body,
        grid=(VOCAB // VCH,),
        in_specs=[pl.BlockSpec((VCH, DIM), lambda i: (i, 0))],
        out_specs=pl.BlockSpec((VCH // 2, 2 * DIM), lambda i: (i, 0)),
        out_shape=jax.ShapeDtypeStruct((VOCAB // 2, 2 * DIM), jnp.float32),
    )(w)


def kernel(token_ids, weight):
    idx = (token_ids.astype(jnp.int32) * 2).reshape(2, NW, HALF_SLAB)
    w_out = jnp.copy(weight)
    w128 = lax.optimization_barrier(jnp.pad(weight, ((0, 0), (0, DIM))))
    table = w128.reshape(2 * VOCAB, DIM)
    pair_rows = _sc_gather(idx, table)
    emb_t = _to_out_layout(pair_rows)
    emb = jnp.transpose(emb_t, (2, 0, 1))
    return emb, w_out


# speedup vs baseline: 1.0116x; 1.0116x over previous
import jax
import jax.numpy as jnp
from jax import lax
from jax.experimental import pallas as pl
from jax.experimental.pallas import tpu as pltpu
from jax.experimental.pallas import tpu_sc as plsc

VOCAB = 1_000_000
DIM = 64
BATCH = 4096
HALF = BATCH // 2
HIST = 200
N = BATCH * HIST

NC, NS = 2, 16
NW = NC * NS
PAIRS_PER_W = HALF // NW
ROWS_PER_W = N // NW
HALF_SLAB = PAIRS_PER_W * HIST
NBANK = 2
STEPS = PAIRS_PER_W - NBANK


def _gather_body(idx_hbm, table_hbm, out_hbm, idx_v, rows_v, gsems, ssems):
    c = lax.axis_index("c")
    s = lax.axis_index("s")
    wid = s * NC + c
    pair0 = wid * PAIRS_PER_W

    pltpu.sync_copy(idx_hbm.at[0, wid], idx_v.at[pl.ds(0, HALF_SLAB)])
    pltpu.sync_copy(idx_hbm.at[1, wid], idx_v.at[pl.ds(HALF_SLAB, HALF_SLAB)])

    def gather(p, g, half):
        return pltpu.make_async_copy(
            table_hbm.at[idx_v.at[pl.ds(half * HALF_SLAB + g * HIST, HIST)]],
            rows_v.at[p, half], gsems.at[p, half])

    def store(p, g, half):
        return pltpu.make_async_copy(
            rows_v.at[p, half],
            out_hbm.at[pair0 + g, :, pl.ds(half * DIM, DIM)],
            ssems.at[p, half])

    for p in range(NBANK):
        gather(p, p, 0).start()
        gather(p, p, 1).start()

    def step(g, _):
        for p in range(NBANK):
            grp = g * NBANK + p
            for half in range(2):
                gather(p, grp, half).wait()
                store(p, grp, half).start()
            for half in range(2):
                store(p, grp, half).wait()
                gather(p, grp + NBANK, half).start()
        return _

    lax.fori_loop(0, STEPS // NBANK, step, None)

    for p in range(NBANK):
        grp = STEPS + p
        for half in range(2):
            gather(p, grp, half).wait()
            store(p, grp, half).start()
    for p in range(NBANK):
        for half in range(2):
            store(p, STEPS + p, half).wait()


@jax.jit
def _sc_gather(idx, table):
    mesh = plsc.VectorSubcoreMesh(
        core_axis_name="c", subcore_axis_name="s",
        num_cores=NC, num_subcores=NS)
    k = pl.kernel(
        _gather_body,
        out_type=jax.ShapeDtypeStruct((HALF, HIST, 2 * DIM), jnp.float32),
        mesh=mesh,
        compiler_params=pltpu.CompilerParams(use_tc_tiling_on_sc=False),
        scratch_types=[
            pltpu.VMEM((ROWS_PER_W,), jnp.int32),
            pltpu.VMEM((NBANK, 2, HIST, DIM), jnp.float32),
            pltpu.SemaphoreType.DMA((NBANK, 2)),
            pltpu.SemaphoreType.DMA((NBANK, 2)),
        ],
    )
    return k(idx, table)


HCH = 8


def _t2_body(in_ref, out_ref):
    def body(hh, _):
        t = in_ref[:, hh, :]
        tt = t.T
        v = tt.reshape(2, DIM, HALF).transpose(1, 0, 2)
        out_ref[hh] = v.reshape(DIM, BATCH)
        return _
    lax.fori_loop(0, HCH, body, None)


@jax.jit
def _to_out_layout(pair_rows):
    return pl.pallas_call(
        _t2_body,
        grid=(HIST // HCH,),
        in_specs=[pl.BlockSpec((HALF, HCH, 2 * DIM), lambda i: (0, i, 0))],
        out_specs=pl.BlockSpec((HCH, DIM, BATCH), lambda i: (i, 0, 0)),
        out_shape=jax.ShapeDtypeStruct((HIST, DIM, BATCH), jnp.float32),
    )(pair_rows)


VCH = 8000


def _t1_body(in_ref, out_ref):
    t3 = in_ref[...].reshape(VCH // 2, 2, DIM)
    out_ref[:, :DIM] = t3[:, 0, :]
    out_ref[:, DIM:] = t3[:, 1, :]


@jax.jit
def _to_table_layout(w):
    return pl.pallas_call(
        _t1_body,
        grid=(VOCAB // VCH,),
        in_specs=[pl.BlockSpec((VCH, DIM), lambda i: (i, 0))],
        out_specs=pl.BlockSpec((VCH // 2, 2 * DIM), lambda i: (i, 0)),
        out_shape=jax.ShapeDtypeStruct((VOCAB // 2, 2 * DIM), jnp.float32),
    )(w)


def kernel(token_ids, weight):
    idx = token_ids.astype(jnp.int32).reshape(2, NW, HALF_SLAB)
    w_out, idx = lax.optimization_barrier((jnp.copy(weight), idx))
    w_pairs = lax.optimization_barrier(weight.reshape(VOCAB // 2, 2 * DIM))
    table = w_pairs.reshape(VOCAB, DIM)
    pair_rows = _sc_gather(idx, table)
    emb_t = _to_out_layout(pair_rows)
    emb = jnp.transpose(emb_t, (2, 0, 1))
    return emb, w_out


# speedup vs baseline: 1.0415x; 1.0296x over previous
import jax
import jax.numpy as jnp
from jax import lax
from jax.experimental import pallas as pl
from jax.experimental.pallas import tpu as pltpu
from jax.experimental.pallas import tpu_sc as plsc

VOCAB = 1_000_000
DIM = 64
BATCH = 4096
HALF = BATCH // 2
HIST = 200
N = BATCH * HIST

NC, NS = 2, 16
NW = NC * NS
PAIRS_PER_W = HALF // NW
ROWS_PER_W = N // NW
HALF_SLAB = PAIRS_PER_W * HIST
NBANK = 2
STEPS = PAIRS_PER_W - NBANK


def _gather_body(idx_hbm, table_hbm, out_hbm, idx_v, rows_v, gsems, ssems):
    c = lax.axis_index("c")
    s = lax.axis_index("s")
    wid = s * NC + c
    pair0 = wid * PAIRS_PER_W

    pltpu.sync_copy(idx_hbm.at[0, wid], idx_v.at[pl.ds(0, HALF_SLAB)])
    pltpu.sync_copy(idx_hbm.at[1, wid], idx_v.at[pl.ds(HALF_SLAB, HALF_SLAB)])

    def gather(p, g, half):
        return pltpu.make_async_copy(
            table_hbm.at[idx_v.at[pl.ds(half * HALF_SLAB + g * HIST, HIST)]],
            rows_v.at[p, half], gsems.at[p, half])

    def store(p, g, half):
        return pltpu.make_async_copy(
            rows_v.at[p, half],
            out_hbm.at[pair0 + g, :, pl.ds(half * DIM, DIM)],
            ssems.at[p, half])

    for p in range(NBANK):
        gather(p, p, 0).start()
        gather(p, p, 1).start()

    def step(g, _):
        for p in range(NBANK):
            grp = g * NBANK + p
            for half in range(2):
                gather(p, grp, half).wait()
                store(p, grp, half).start()
            for half in range(2):
                store(p, grp, half).wait()
                gather(p, grp + NBANK, half).start()
        return _

    lax.fori_loop(0, STEPS // NBANK, step, None)

    for p in range(NBANK):
        grp = STEPS + p
        for half in range(2):
            gather(p, grp, half).wait()
            store(p, grp, half).start()
    for p in range(NBANK):
        for half in range(2):
            store(p, STEPS + p, half).wait()


@jax.jit
def _sc_gather(idx, table):
    mesh = plsc.VectorSubcoreMesh(
        core_axis_name="c", subcore_axis_name="s",
        num_cores=NC, num_subcores=NS)
    k = pl.kernel(
        _gather_body,
        out_type=jax.ShapeDtypeStruct((HALF, HIST, 2 * DIM), jnp.float32),
        mesh=mesh,
        compiler_params=pltpu.CompilerParams(use_tc_tiling_on_sc=False),
        scratch_types=[
            pltpu.VMEM((ROWS_PER_W,), jnp.int32),
            pltpu.VMEM((NBANK, 2, HIST, DIM), jnp.float32),
            pltpu.SemaphoreType.DMA((NBANK, 2)),
            pltpu.SemaphoreType.DMA((NBANK, 2)),
        ],
    )
    return k(idx, table)


HCH = 8


def _t2_body(in_ref, out_ref):
    def body(hh, _):
        t = in_ref[:, hh, :]
        tt = t.T
        v = tt.reshape(2, DIM, HALF).transpose(1, 0, 2)
        out_ref[hh] = v.reshape(DIM, BATCH)
        return _
    lax.fori_loop(0, HCH, body, None)


@jax.jit
def _to_out_layout(pair_rows):
    return pl.pallas_call(
        _t2_body,
        grid=(HIST // HCH,),
        in_specs=[pl.BlockSpec((HALF, HCH, 2 * DIM), lambda i: (0, i, 0))],
        out_specs=pl.BlockSpec((HCH, DIM, BATCH), lambda i: (i, 0, 0)),
        out_shape=jax.ShapeDtypeStruct((HIST, DIM, BATCH), jnp.float32),
    )(pair_rows)


VCH = 20000


def _t1_body(in_ref, out_ref):
    t3 = in_ref[...].reshape(VCH // 2, 2, DIM)
    out_ref[:, :DIM] = t3[:, 0, :]
    out_ref[:, DIM:] = t3[:, 1, :]


@jax.jit
def _to_table_layout(w):
    return pl.pallas_call(
        _t1_body,
        grid=(VOCAB // VCH,),
        in_specs=[pl.BlockSpec((VCH, DIM), lambda i: (i, 0))],
        out_specs=pl.BlockSpec((VCH // 2, 2 * DIM), lambda i: (i, 0)),
        out_shape=jax.ShapeDtypeStruct((VOCAB // 2, 2 * DIM), jnp.float32),
    )(w)


def kernel(token_ids, weight):
    idx = token_ids.astype(jnp.int32).reshape(2, NW, HALF_SLAB)
    w_out, idx = lax.optimization_barrier((jnp.copy(weight), idx))
    w_tiled = lax.optimization_barrier(weight.reshape(VOCAB // 8, 8, DIM))
    w_pairs = _to_table_layout(w_tiled.reshape(VOCAB, DIM))
    table = w_pairs.reshape(VOCAB, DIM)
    pair_rows = _sc_gather(idx, table)
    emb_t = _to_out_layout(pair_rows)
    emb = jnp.transpose(emb_t, (2, 0, 1))
    return emb, w_out
